# Initial kernel scaffold; baseline (speedup 1.0000x reference)
import jax
import jax.numpy as jnp
from jax.experimental import pallas as pl


def kernel(x, adj, e, Wn, bn, We, be, Wm1, bm1, Wu1, bu1, g1, bln1, Wm2, bm2, Wu2, bu2, g2, bln2):
    raise NotImplementedError("write your pallas kernel here")


# speedup vs baseline: 2.0179x; 2.0179x over previous
import functools

import jax
import jax.numpy as jnp
from jax import lax
from jax.experimental import pallas as pl
from jax.experimental.pallas import tpu as pltpu
from jax.experimental.pallas import tpu_sc as plsc

D = 256
H = 128
EPS = 1e-15
MAXNORM = 1.0 - 1e-5
SUBCORES = 16


def _norm(x):
    return jnp.clip(jnp.sqrt(jnp.sum(x * x, axis=-1, keepdims=True)), EPS, None)

def _artanh(x):
    x = jnp.clip(x, -1.0 + 1e-7, 1.0 - 1e-7)
    return 0.5 * jnp.log((1.0 + x) / (1.0 - x))

def _proj(x):
    n = _norm(x)
    return jnp.where(n > MAXNORM, x / n * MAXNORM, x)

def _expmap0(u):
    n = _norm(u)
    return _proj(jnp.tanh(n) * u / n)

def _logmap0(x):
    n = _norm(x)
    return _artanh(n) * x / n

def _mobius_add(x, y):
    x2 = jnp.sum(x * x, -1, keepdims=True)
    y2 = jnp.sum(y * y, -1, keepdims=True)
    xy = jnp.sum(x * y, -1, keepdims=True)
    num = (1 + 2 * xy + y2) * x + (1 - x2) * y
    den = 1 + 2 * xy + x2 * y2
    return num / jnp.clip(den, EPS, None)

def _poincare_linear_rows(x, wT, b_row):
    xn = _norm(x)
    mx = jnp.dot(x, wT, preferred_element_type=jnp.float32)
    mxn = _norm(mx)
    h = _proj(jnp.tanh(mxn / xn * _artanh(xn)) * mx / mxn)
    hb = _expmap0(b_row)
    return _proj(_mobius_add(h, jnp.broadcast_to(hb, h.shape)))

def _layernorm(x, g, b):
    mu = jnp.mean(x, -1, keepdims=True)
    var = jnp.var(x, -1, keepdims=True)
    return (x - mu) / jnp.sqrt(var + 1e-5) * g + b


def _node_pre_body(x_ref, wnT_ref, bn_ref, wm1tT_ref, bm1_ref, xt_ref, a_ref):
    h = _poincare_linear_rows(x_ref[...], wnT_ref[...], bn_ref[...])
    xt = _logmap0(h)
    xt_ref[...] = xt
    a = jnp.dot(xt, wm1tT_ref[...], preferred_element_type=jnp.float32) + bm1_ref[...]
    a_ref[0] = a[:, :H]
    a_ref[1] = a[:, H:]


def _edge_pre_body(e_ref, weT_ref, be_ref, wmbot_ref, eb1_ref, eb2_ref):
    h = _poincare_linear_rows(e_ref[...], weT_ref[...], be_ref[...])
    et = _logmap0(h)
    eb = jnp.dot(et, wmbot_ref[...], preferred_element_type=jnp.float32)
    eb1_ref[0] = eb[:, 0 * H:1 * H]
    eb1_ref[1] = eb[:, 1 * H:2 * H]
    eb2_ref[0] = eb[:, 2 * H:3 * H]
    eb2_ref[1] = eb[:, 3 * H:4 * H]


def _mid_body(xt_ref, agg_ref, deg_ref, wu_ref, bu_ref, g_ref, bln_ref,
              wmtT_ref, bm_ref, xt2_ref, a2_ref):
    agg = jnp.concatenate([agg_ref[0], agg_ref[1]], axis=1)
    deg = jnp.clip(deg_ref[0, :, 0:1] + deg_ref[1, :, 0:1], 1.0, None)
    out_t = jnp.dot(xt_ref[...] + agg / deg, wu_ref[...],
                    preferred_element_type=jnp.float32) + bu_ref[...]
    out_t = jax.nn.relu(_layernorm(out_t, g_ref[...], bln_ref[...]))
    h = _proj(_expmap0(out_t))
    xt2 = _logmap0(h)
    xt2_ref[...] = xt2
    a2 = jnp.dot(xt2, wmtT_ref[...], preferred_element_type=jnp.float32) + bm_ref[...]
    a2_ref[0] = a2[:, :H]
    a2_ref[1] = a2[:, H:]


def _final_body(xt_ref, agg_ref, deg_ref, wu_ref, bu_ref, g_ref,
                bln_ref, out_ref):
    agg = jnp.concatenate([agg_ref[0], agg_ref[1]], axis=1)
    deg = jnp.clip(deg_ref[0, :, 0:1] + deg_ref[1, :, 0:1], 1.0, None)
    out_t = jnp.dot(xt_ref[...] + agg / deg, wu_ref[...],
                    preferred_element_type=jnp.float32) + bu_ref[...]
    out_t = jax.nn.relu(_layernorm(out_t, g_ref[...], bln_ref[...]))
    out_ref[...] = _proj(_expmap0(out_t))


def _row_spec(r, cols):
    return pl.BlockSpec((r, cols), lambda i: (i, 0))

def _full_spec(shape):
    nd = len(shape)
    return pl.BlockSpec(shape, lambda i: (0,) * nd)

def _half_spec(r, w=H):
    return pl.BlockSpec((2, r, w), lambda i: (0, i, 0))


@functools.lru_cache(maxsize=None)
def _make_sc_layer(with_deg, n, e):
    edges_per_sub = e // SUBCORES
    ch = 80 if edges_per_sub % 80 == 0 else edges_per_sub
    chunks = edges_per_sub // ch
    edges_per_w = e // (2 * SUBCORES)
    ch2 = max(d for d in range(8, 129, 8) if edges_per_w % d == 0)
    chunks2 = edges_per_w // ch2
    zch = 32
    rows_per_sub = -(-(n // SUBCORES) // zch) * zch
    n_pad = rows_per_sub * SUBCORES
    zreps = rows_per_sub // zch

    mesh = plsc.VectorSubcoreMesh(core_axis_name="c", subcore_axis_name="s",
                                  num_cores=2, num_subcores=SUBCORES)

    agg_t = jax.ShapeDtypeStruct((2 * n_pad, H), jnp.float32)
    scratch = [
        pltpu.VMEM((ch,), jnp.int32),
        pltpu.VMEM((ch,), jnp.int32),
        pltpu.VMEM((ch,), jnp.int32),
        pltpu.VMEM((ch, H), jnp.float32),
        pltpu.VMEM((ch, H), jnp.float32),
        pltpu.VMEM((ch, H), jnp.float32),
        pltpu.VMEM((zch, H), jnp.float32),
        pltpu.VMEM_SHARED((n_pad, H), jnp.float32),
        pltpu.SemaphoreType.DMA,
    ]
    if with_deg:
        scratch += [
            pltpu.VMEM((ch2,), jnp.int32),
            pltpu.VMEM((ch2, H), jnp.float32),
        ]

    @functools.partial(
        pl.kernel,
        out_type=(agg_t, agg_t) if with_deg else agg_t,
        mesh=mesh,
        scratch_types=scratch,
    )
    def sc_layer(acat, ebcat, src, dst, *refs):
        if with_deg:
            (out, outd, src_v, dst_v, gidx_v, a_v, eb_v, msg_v, z_v, agg_sp,
             sem, dst2_v, ones_v) = refs
        else:
            (out, src_v, dst_v, gidx_v, a_v, eb_v, msg_v, z_v, agg_sp,
             sem) = refs
        c = lax.axis_index("c")
        s = lax.axis_index("s")
        row0 = s * rows_per_sub
        cN = c * n
        cNp = c * n_pad
        cE = c * e

        def zrow(r, carry):
            for j in range(H // 16):
                z_v[r, pl.ds(j * 16, 16)] = jnp.zeros((16,), jnp.float32)
            return carry
        lax.fori_loop(0, zch, zrow, 0)

        def zcopy(t, carry):
            pltpu.sync_copy(z_v, agg_sp.at[pl.ds(row0 + t * zch, zch)])
            return carry
        lax.fori_loop(0, zreps, zcopy, 0)

        plsc.subcore_barrier()

        def chunk(k, carry):
            base = s * edges_per_sub + k * ch
            pltpu.sync_copy(src.at[pl.ds(base, ch)], src_v)
            pltpu.sync_copy(dst.at[pl.ds(base, ch)], dst_v)
            for j in range(ch // 16):
                sl = pl.ds(j * 16, 16)
                gidx_v[sl] = src_v[sl] + cN
            pltpu.async_copy(acat.at[gidx_v], a_v, sem).wait()
            pltpu.sync_copy(ebcat.at[pl.ds(cE + base, ch)], eb_v)

            def mrow(r, rc):
                for j in range(H // 16):
                    sl = pl.ds(j * 16, 16)
                    msg_v[r, sl] = jnp.maximum(a_v[r, sl] + eb_v[r, sl], 0.0)
                return rc
            lax.fori_loop(0, ch, mrow, 0)

            pltpu.sync_copy(msg_v, agg_sp.at[dst_v], add=True)
            return carry

        lax.fori_loop(0, chunks, chunk, 0)

        plsc.subcore_barrier()
        pltpu.sync_copy(agg_sp.at[pl.ds(row0, rows_per_sub)],
                        out.at[pl.ds(cNp + row0, rows_per_sub)])

        if with_deg:
            lax.fori_loop(0, zreps, zcopy, 0)

            def onerow(r, carry):
                for j in range(H // 16):
                    ones_v[r, pl.ds(j * 16, 16)] = jnp.ones((16,), jnp.float32)
                return carry
            lax.fori_loop(0, ch2, onerow, 0)

            plsc.subcore_barrier()
            w0 = (c * SUBCORES + s) * edges_per_w

            def dchunk(k, carry):
                pltpu.sync_copy(dst.at[pl.ds(w0 + k * ch2, ch2)], dst2_v)
                pltpu.sync_copy(ones_v, agg_sp.at[dst2_v], add=True)
                return carry
            lax.fori_loop(0, chunks2, dchunk, 0)

            plsc.subcore_barrier()
            pltpu.sync_copy(agg_sp.at[pl.ds(row0, rows_per_sub)],
                            outd.at[pl.ds(cNp + row0, rows_per_sub)])

    return sc_layer


def kernel(x, adj, e, Wn, bn, We, be, Wm1, bm1, Wu1, bu1, g1, bln1,
           Wm2, bm2, Wu2, bu2, g2, bln2):
    f32 = jnp.float32
    n = x.shape[0]
    ne = e.shape[0]
    rn = 1000 if n % 1000 == 0 else n
    re_ = 640 if ne % 640 == 0 else ne
    n_pad = (-(-(n // SUBCORES) // 32) * 32) * SUBCORES

    src = adj[0]
    dst = adj[1]

    wnT = Wn.T
    weT = We.T
    wm1t = Wm1[:D]
    wm2t = Wm2[:D]
    wmbot = jnp.concatenate([Wm1[D:], Wm2[D:]], axis=1)
    bn_r = bn.reshape(1, D)
    be_r = be.reshape(1, D)
    bm1_r = bm1.reshape(1, D)
    bm2_r = bm2.reshape(1, D)
    bu1_r = bu1.reshape(1, D)
    bu2_r = bu2.reshape(1, D)
    g1_r = g1.reshape(1, D)
    g2_r = g2.reshape(1, D)
    bln1_r = bln1.reshape(1, D)
    bln2_r = bln2.reshape(1, D)

    xt1, a1 = pl.pallas_call(
        _node_pre_body,
        grid=(n // rn,),
        in_specs=[_row_spec(rn, D), _full_spec((D, D)), _full_spec((1, D)),
                  _full_spec((D, D)), _full_spec((1, D))],
        out_specs=[_row_spec(rn, D), _half_spec(rn)],
        out_shape=[jax.ShapeDtypeStruct((n, D), f32),
                   jax.ShapeDtypeStruct((2, n, H), f32)],
    )(x, wnT, bn_r, wm1t, bm1_r)

    eb1, eb2 = pl.pallas_call(
        _edge_pre_body,
        grid=(ne // re_,),
        in_specs=[_row_spec(re_, D), _full_spec((D, D)), _full_spec((1, D)),
                  _full_spec((D, 2 * D))],
        out_specs=[_half_spec(re_), _half_spec(re_)],
        out_shape=[jax.ShapeDtypeStruct((2, ne, H), f32),
                   jax.ShapeDtypeStruct((2, ne, H), f32)],
    )(e, weT, be_r, wmbot)

    agg1, deg1 = _make_sc_layer(True, n, ne)(
        a1.reshape(2 * n, H), eb1.reshape(2 * ne, H), src, dst)
    agg1 = agg1.reshape(2, n_pad, H)
    deg1 = deg1.reshape(2, n_pad, H)

    xt2, a2 = pl.pallas_call(
        _mid_body,
        grid=(n // rn,),
        in_specs=[_row_spec(rn, D), _half_spec(rn, H), _half_spec(rn, H),
                  _full_spec((D, D)), _full_spec((1, D)), _full_spec((1, D)),
                  _full_spec((1, D)), _full_spec((D, D)), _full_spec((1, D))],
        out_specs=[_row_spec(rn, D), _half_spec(rn)],
        out_shape=[jax.ShapeDtypeStruct((n, D), f32),
                   jax.ShapeDtypeStruct((2, n, H), f32)],
    )(xt1, agg1, deg1, Wu1, bu1_r, g1_r, bln1_r, wm2t, bm2_r)

    agg2 = _make_sc_layer(False, n, ne)(
        a2.reshape(2 * n, H), eb2.reshape(2 * ne, H), src, dst)
    agg2 = agg2.reshape(2, n_pad, H)

    out = pl.pallas_call(
        _final_body,
        grid=(n // rn,),
        in_specs=[_row_spec(rn, D), _half_spec(rn, H), _half_spec(rn, H),
                  _full_spec((D, D)), _full_spec((1, D)), _full_spec((1, D)),
                  _full_spec((1, D))],
        out_specs=_row_spec(rn, D),
        out_shape=jax.ShapeDtypeStruct((n, D), f32),
    )(xt2, agg2, deg1, Wu2, bu2_r, g2_r, bln2_r)

    return out


# speedup vs baseline: 2.6669x; 1.3216x over previous
import functools

import jax
import jax.numpy as jnp
from jax import lax
from jax.experimental import pallas as pl
from jax.experimental.pallas import tpu as pltpu
from jax.experimental.pallas import tpu_sc as plsc

D = 256
H = 128
EPS = 1e-15
MAXNORM = 1.0 - 1e-5
SUBCORES = 16


def _norm(x):
    return jnp.clip(jnp.sqrt(jnp.sum(x * x, axis=-1, keepdims=True)), EPS, None)

def _artanh(x):
    x = jnp.clip(x, -1.0 + 1e-7, 1.0 - 1e-7)
    return 0.5 * jnp.log((1.0 + x) / (1.0 - x))

def _proj(x):
    n = _norm(x)
    return jnp.where(n > MAXNORM, x / n * MAXNORM, x)

def _expmap0(u):
    n = _norm(u)
    return _proj(jnp.tanh(n) * u / n)

def _logmap0(x):
    n = _norm(x)
    return _artanh(n) * x / n

def _mobius_add(x, y):
    x2 = jnp.sum(x * x, -1, keepdims=True)
    y2 = jnp.sum(y * y, -1, keepdims=True)
    xy = jnp.sum(x * y, -1, keepdims=True)
    num = (1 + 2 * xy + y2) * x + (1 - x2) * y
    den = 1 + 2 * xy + x2 * y2
    return num / jnp.clip(den, EPS, None)

def _poincare_linear_rows(x, wT, b_row):
    xn = _norm(x)
    mx = jnp.dot(x, wT, preferred_element_type=jnp.float32)
    mxn = _norm(mx)
    h = _proj(jnp.tanh(mxn / xn * _artanh(xn)) * mx / mxn)
    hb = _expmap0(b_row)
    return _proj(_mobius_add(h, jnp.broadcast_to(hb, h.shape)))


def _plin_scalars(xn, mxn, xy0, y2):
    t = jnp.tanh(mxn / xn * _artanh(xn))
    th = jnp.minimum(t, MAXNORM)
    s_h = th / mxn
    x2 = th * th
    xy = s_h * xy0
    a = 1.0 + 2.0 * xy + y2
    b = 1.0 - x2
    den = jnp.clip(1.0 + 2.0 * xy + x2 * y2, EPS, None)
    p = a * s_h / den
    q = b / den
    n2 = jnp.clip(jnp.sqrt(a * a * x2 + 2.0 * a * b * xy + b * b * y2) / den,
                  EPS, None)
    gam = jnp.where(n2 > MAXNORM, MAXNORM / n2, 1.0)
    n3 = jnp.clip(gam * n2, EPS, None)
    lam = _artanh(n3) / n3
    return lam * gam * p, lam * gam * q

def _layernorm(x, g, b):
    mu = jnp.mean(x, -1, keepdims=True)
    var = jnp.var(x, -1, keepdims=True)
    return (x - mu) / jnp.sqrt(var + 1e-5) * g + b


def _node_pre_body(x_ref, w1_ref, hb_ref, hbw_ref, bm1_ref, xt_ref, a_ref):
    x = x_ref[...]
    g = jnp.dot(x, w1_ref[...], preferred_element_type=jnp.float32)
    mx = g[:, :D]
    mxa = g[:, D:]
    hb = hb_ref[...]
    y2 = jnp.sum(hb * hb, -1, keepdims=True)
    xy0 = jnp.sum(mx * hb, -1, keepdims=True)
    P, Q = _plin_scalars(_norm(x), _norm(mx), xy0, y2)
    xt_ref[...] = P * mx + Q * hb
    a = P * mxa + (Q * hbw_ref[...] + bm1_ref[...])
    a_ref[0] = a[:, :H]
    a_ref[1] = a[:, H:]


def _edge_pre_body(e_ref, w1_ref, hb_ref, hbw_ref, eb1_ref, eb2_ref):
    x = e_ref[...]
    g = jnp.dot(x, w1_ref[...], preferred_element_type=jnp.float32)
    mx = g[:, :D]
    mxw = g[:, D:]
    hb = hb_ref[...]
    y2 = jnp.sum(hb * hb, -1, keepdims=True)
    xy0 = jnp.sum(mx * hb, -1, keepdims=True)
    P, Q = _plin_scalars(_norm(x), _norm(mx), xy0, y2)
    eb = P * mxw + Q * hbw_ref[...]
    eb1_ref[0] = eb[:, 0 * H:1 * H]
    eb1_ref[1] = eb[:, 1 * H:2 * H]
    eb2_ref[0] = eb[:, 2 * H:3 * H]
    eb2_ref[1] = eb[:, 3 * H:4 * H]


def _mid_body(xt_ref, agg_ref, deg_ref, wu_ref, bu_ref, g_ref, bln_ref,
              wmtT_ref, bm_ref, xt2_ref, a2_ref):
    agg = jnp.concatenate([agg_ref[0], agg_ref[1]], axis=1)
    deg = jnp.clip(deg_ref[0, :, 0:1] + deg_ref[1, :, 0:1], 1.0, None)
    out_t = jnp.dot(xt_ref[...] + agg / deg, wu_ref[...],
                    preferred_element_type=jnp.float32) + bu_ref[...]
    z = jax.nn.relu(_layernorm(out_t, g_ref[...], bln_ref[...]))
    nz = _norm(z)
    r = _artanh(jnp.minimum(jnp.tanh(nz), MAXNORM)) / nz
    xt2_ref[...] = r * z
    a2 = r * jnp.dot(z, wmtT_ref[...], preferred_element_type=jnp.float32) \
        + bm_ref[...]
    a2_ref[0] = a2[:, :H]
    a2_ref[1] = a2[:, H:]


def _final_body(xt_ref, agg_ref, deg_ref, wu_ref, bu_ref, g_ref,
                bln_ref, out_ref):
    agg = jnp.concatenate([agg_ref[0], agg_ref[1]], axis=1)
    deg = jnp.clip(deg_ref[0, :, 0:1] + deg_ref[1, :, 0:1], 1.0, None)
    out_t = jnp.dot(xt_ref[...] + agg / deg, wu_ref[...],
                    preferred_element_type=jnp.float32) + bu_ref[...]
    z = jax.nn.relu(_layernorm(out_t, g_ref[...], bln_ref[...]))
    nz = _norm(z)
    out_ref[...] = (jnp.minimum(jnp.tanh(nz), MAXNORM) / nz) * z


def _row_spec(r, cols):
    return pl.BlockSpec((r, cols), lambda i: (i, 0))

def _full_spec(shape):
    nd = len(shape)
    return pl.BlockSpec(shape, lambda i: (0,) * nd)

def _half_spec(r, w=H):
    return pl.BlockSpec((2, r, w), lambda i: (0, i, 0))


@functools.lru_cache(maxsize=None)
def _make_sc_layer(with_deg, n, e):
    edges_per_sub = e // SUBCORES
    ch = 80 if edges_per_sub % 80 == 0 else edges_per_sub
    chunks = edges_per_sub // ch
    edges_per_w = e // (2 * SUBCORES)
    ch2 = max(d for d in range(8, 129, 8) if edges_per_w % d == 0)
    chunks2 = edges_per_w // ch2
    zch = 32
    rows_per_sub = -(-(n // SUBCORES) // zch) * zch
    n_pad = rows_per_sub * SUBCORES
    zreps = rows_per_sub // zch

    mesh = plsc.VectorSubcoreMesh(core_axis_name="c", subcore_axis_name="s",
                                  num_cores=2, num_subcores=SUBCORES)

    agg_t = jax.ShapeDtypeStruct((2 * n_pad, H), jnp.float32)
    scratch = [
        pltpu.VMEM((ch,), jnp.int32),
        pltpu.VMEM((ch,), jnp.int32),
        pltpu.VMEM((ch,), jnp.int32),
        pltpu.VMEM((ch, H), jnp.float32),
        pltpu.VMEM((ch, H), jnp.float32),
        pltpu.VMEM((ch, H), jnp.float32),
        pltpu.VMEM((zch, H), jnp.float32),
        pltpu.VMEM_SHARED((n_pad, H), jnp.float32),
        pltpu.SemaphoreType.DMA,
        pltpu.SemaphoreType.DMA,
        pltpu.SemaphoreType.DMA,
        pltpu.SemaphoreType.DMA,
    ]
    if with_deg:
        scratch += [
            pltpu.VMEM((ch2,), jnp.int32),
            pltpu.VMEM((ch2, H), jnp.float32),
        ]

    @functools.partial(
        pl.kernel,
        out_type=(agg_t, agg_t) if with_deg else agg_t,
        mesh=mesh,
        scratch_types=scratch,
    )
    def sc_layer(acat, ebcat, src, dst, *refs):
        if with_deg:
            (out, outd, src_v, dst_v, gidx_v, a_v, eb_v, msg_v, z_v, agg_sp,
             sem_src, sem_dst, sem_eb, sem_g, dst2_v, ones_v) = refs
        else:
            (out, src_v, dst_v, gidx_v, a_v, eb_v, msg_v, z_v, agg_sp,
             sem_src, sem_dst, sem_eb, sem_g) = refs
        c = lax.axis_index("c")
        s = lax.axis_index("s")
        row0 = s * rows_per_sub
        cN = c * n
        cNp = c * n_pad
        cE = c * e

        def zrow(r, carry):
            for j in range(H // 16):
                z_v[r, pl.ds(j * 16, 16)] = jnp.zeros((16,), jnp.float32)
            return carry
        lax.fori_loop(0, zch, zrow, 0)

        def zcopy(t, carry):
            pltpu.sync_copy(z_v, agg_sp.at[pl.ds(row0 + t * zch, zch)])
            return carry
        lax.fori_loop(0, zreps, zcopy, 0)

        plsc.subcore_barrier()

        def chunk(k, carry):
            base = s * edges_per_sub + k * ch
            h_eb = pltpu.async_copy(ebcat.at[pl.ds(cE + base, ch)], eb_v,
                                    sem_eb)
            h_dst = pltpu.async_copy(dst.at[pl.ds(base, ch)], dst_v, sem_dst)
            h_src = pltpu.async_copy(src.at[pl.ds(base, ch)], src_v, sem_src)
            h_src.wait()
            for j in range(ch // 16):
                sl = pl.ds(j * 16, 16)
                gidx_v[sl] = src_v[sl] + cN
            h_g = pltpu.async_copy(acat.at[gidx_v], a_v, sem_g)
            h_g.wait()
            h_eb.wait()

            def mrow(r, rc):
                for j in range(H // 16):
                    sl = pl.ds(j * 16, 16)
                    msg_v[r, sl] = jnp.maximum(a_v[r, sl] + eb_v[r, sl], 0.0)
                return rc
            lax.fori_loop(0, ch, mrow, 0)

            h_dst.wait()
            pltpu.sync_copy(msg_v, agg_sp.at[dst_v], add=True)
            return carry

        lax.fori_loop(0, chunks, chunk, 0)

        plsc.subcore_barrier()
        pltpu.sync_copy(agg_sp.at[pl.ds(row0, rows_per_sub)],
                        out.at[pl.ds(cNp + row0, rows_per_sub)])

        if with_deg:
            lax.fori_loop(0, zreps, zcopy, 0)

            def onerow(r, carry):
                for j in range(H // 16):
                    ones_v[r, pl.ds(j * 16, 16)] = jnp.ones((16,), jnp.float32)
                return carry
            lax.fori_loop(0, ch2, onerow, 0)

            plsc.subcore_barrier()
            w0 = (c * SUBCORES + s) * edges_per_w

            def dchunk(k, carry):
                pltpu.sync_copy(dst.at[pl.ds(w0 + k * ch2, ch2)], dst2_v)
                pltpu.sync_copy(ones_v, agg_sp.at[dst2_v], add=True)
                return carry
            lax.fori_loop(0, chunks2, dchunk, 0)

            plsc.subcore_barrier()
            pltpu.sync_copy(agg_sp.at[pl.ds(row0, rows_per_sub)],
                            outd.at[pl.ds(cNp + row0, rows_per_sub)])

    return sc_layer


def kernel(x, adj, e, Wn, bn, We, be, Wm1, bm1, Wu1, bu1, g1, bln1,
           Wm2, bm2, Wu2, bu2, g2, bln2):
    f32 = jnp.float32
    n = x.shape[0]
    ne = e.shape[0]
    rn = 1000 if n % 1000 == 0 else n
    re_ = 640 if ne % 640 == 0 else ne
    n_pad = (-(-(n // SUBCORES) // 32) * 32) * SUBCORES

    src = adj[0]
    dst = adj[1]

    wm1t = Wm1[:D]
    wm2t = Wm2[:D]
    wmbot = jnp.concatenate([Wm1[D:], Wm2[D:]], axis=1)
    wnT = Wn.T
    weT = We.T
    w1n = jnp.concatenate([wnT, wnT @ wm1t], axis=1)
    w1e = jnp.concatenate([weT, weT @ wmbot], axis=1)
    hb_n = _expmap0(bn.reshape(1, D))
    hb_e = _expmap0(be.reshape(1, D))
    hbw_n = hb_n @ wm1t
    hbw_e = hb_e @ wmbot
    bm1_r = bm1.reshape(1, D)
    bm2_r = bm2.reshape(1, D)
    bu1_r = bu1.reshape(1, D)
    bu2_r = bu2.reshape(1, D)
    g1_r = g1.reshape(1, D)
    g2_r = g2.reshape(1, D)
    bln1_r = bln1.reshape(1, D)
    bln2_r = bln2.reshape(1, D)

    xt1, a1 = pl.pallas_call(
        _node_pre_body,
        grid=(n // rn,),
        in_specs=[_row_spec(rn, D), _full_spec((D, 2 * D)), _full_spec((1, D)),
                  _full_spec((1, D)), _full_spec((1, D))],
        out_specs=[_row_spec(rn, D), _half_spec(rn)],
        out_shape=[jax.ShapeDtypeStruct((n, D), f32),
                   jax.ShapeDtypeStruct((2, n, H), f32)],
    )(x, w1n, hb_n, hbw_n, bm1_r)

    eb1, eb2 = pl.pallas_call(
        _edge_pre_body,
        grid=(ne // re_,),
        in_specs=[_row_spec(re_, D), _full_spec((D, 3 * D)), _full_spec((1, D)),
                  _full_spec((1, 2 * D))],
        out_specs=[_half_spec(re_), _half_spec(re_)],
        out_shape=[jax.ShapeDtypeStruct((2, ne, H), f32),
                   jax.ShapeDtypeStruct((2, ne, H), f32)],
    )(e, w1e, hb_e, hbw_e)

    agg1, deg1 = _make_sc_layer(True, n, ne)(
        a1.reshape(2 * n, H), eb1.reshape(2 * ne, H), src, dst)
    agg1 = agg1.reshape(2, n_pad, H)
    deg1 = deg1.reshape(2, n_pad, H)

    xt2, a2 = pl.pallas_call(
        _mid_body,
        grid=(n // rn,),
        in_specs=[_row_spec(rn, D), _half_spec(rn, H), _half_spec(rn, H),
                  _full_spec((D, D)), _full_spec((1, D)), _full_spec((1, D)),
                  _full_spec((1, D)), _full_spec((D, D)), _full_spec((1, D))],
        out_specs=[_row_spec(rn, D), _half_spec(rn)],
        out_shape=[jax.ShapeDtypeStruct((n, D), f32),
                   jax.ShapeDtypeStruct((2, n, H), f32)],
    )(xt1, agg1, deg1, Wu1, bu1_r, g1_r, bln1_r, wm2t, bm2_r)

    agg2 = _make_sc_layer(False, n, ne)(
        a2.reshape(2 * n, H), eb2.reshape(2 * ne, H), src, dst)
    agg2 = agg2.reshape(2, n_pad, H)

    out = pl.pallas_call(
        _final_body,
        grid=(n // rn,),
        in_specs=[_row_spec(rn, D), _half_spec(rn, H), _half_spec(rn, H),
                  _full_spec((D, D)), _full_spec((1, D)), _full_spec((1, D)),
                  _full_spec((1, D))],
        out_specs=_row_spec(rn, D),
        out_shape=jax.ShapeDtypeStruct((n, D), f32),
    )(xt2, agg2, deg1, Wu2, bu2_r, g2_r, bln2_r)

    return out


# speedup vs baseline: 2.6828x; 1.0060x over previous
import functools

import jax
import jax.numpy as jnp
from jax import lax
from jax.experimental import pallas as pl
from jax.experimental.pallas import tpu as pltpu
from jax.experimental.pallas import tpu_sc as plsc

D = 256
H = 128
EPS = 1e-15
MAXNORM = 1.0 - 1e-5
SUBCORES = 16


def _norm(x):
    return jnp.clip(jnp.sqrt(jnp.sum(x * x, axis=-1, keepdims=True)), EPS, None)

def _artanh(x):
    x = jnp.clip(x, -1.0 + 1e-7, 1.0 - 1e-7)
    return 0.5 * jnp.log((1.0 + x) / (1.0 - x))

def _proj(x):
    n = _norm(x)
    return jnp.where(n > MAXNORM, x / n * MAXNORM, x)

def _expmap0(u):
    n = _norm(u)
    return _proj(jnp.tanh(n) * u / n)

def _logmap0(x):
    n = _norm(x)
    return _artanh(n) * x / n

def _mobius_add(x, y):
    x2 = jnp.sum(x * x, -1, keepdims=True)
    y2 = jnp.sum(y * y, -1, keepdims=True)
    xy = jnp.sum(x * y, -1, keepdims=True)
    num = (1 + 2 * xy + y2) * x + (1 - x2) * y
    den = 1 + 2 * xy + x2 * y2
    return num / jnp.clip(den, EPS, None)

def _poincare_linear_rows(x, wT, b_row):
    xn = _norm(x)
    mx = jnp.dot(x, wT, preferred_element_type=jnp.float32)
    mxn = _norm(mx)
    h = _proj(jnp.tanh(mxn / xn * _artanh(xn)) * mx / mxn)
    hb = _expmap0(b_row)
    return _proj(_mobius_add(h, jnp.broadcast_to(hb, h.shape)))


def _plin_scalars(xn, mxn, xy0, y2):
    t = jnp.tanh(mxn / xn * _artanh(xn))
    th = jnp.minimum(t, MAXNORM)
    s_h = th / mxn
    x2 = th * th
    xy = s_h * xy0
    a = 1.0 + 2.0 * xy + y2
    b = 1.0 - x2
    den = jnp.clip(1.0 + 2.0 * xy + x2 * y2, EPS, None)
    p = a * s_h / den
    q = b / den
    n2 = jnp.clip(jnp.sqrt(a * a * x2 + 2.0 * a * b * xy + b * b * y2) / den,
                  EPS, None)
    gam = jnp.where(n2 > MAXNORM, MAXNORM / n2, 1.0)
    n3 = jnp.clip(gam * n2, EPS, None)
    lam = _artanh(n3) / n3
    return lam * gam * p, lam * gam * q

def _layernorm(x, g, b):
    mu = jnp.mean(x, -1, keepdims=True)
    var = jnp.var(x, -1, keepdims=True)
    return (x - mu) / jnp.sqrt(var + 1e-5) * g + b


def _node_pre_body(x_ref, w1_ref, hb_ref, hbw_ref, bm1_ref, xt_ref, a_ref):
    x = x_ref[...]
    g = jnp.dot(x, w1_ref[...], preferred_element_type=jnp.float32)
    mx = g[:, :D]
    mxa = g[:, D:]
    hb = hb_ref[...]
    y2 = jnp.sum(hb * hb, -1, keepdims=True)
    xy0 = jnp.sum(mx * hb, -1, keepdims=True)
    P, Q = _plin_scalars(_norm(x), _norm(mx), xy0, y2)
    xt_ref[...] = P * mx + Q * hb
    a = P * mxa + (Q * hbw_ref[...] + bm1_ref[...])
    a_ref[0] = a[:, :H]
    a_ref[1] = a[:, H:]


def _edge_pre_body(e_ref, w1_ref, hb_ref, hbw_ref, eb1_ref, eb2_ref):
    x = e_ref[...]
    g = jnp.dot(x.astype(jnp.bfloat16), w1_ref[...],
                preferred_element_type=jnp.float32)
    mx = g[:, :D]
    mxw = g[:, D:]
    hb = hb_ref[...]
    y2 = jnp.sum(hb * hb, -1, keepdims=True)
    xy0 = jnp.sum(mx * hb, -1, keepdims=True)
    P, Q = _plin_scalars(_norm(x), _norm(mx), xy0, y2)
    eb = P * mxw + Q * hbw_ref[...]
    eb1_ref[0] = eb[:, 0 * H:1 * H]
    eb1_ref[1] = eb[:, 1 * H:2 * H]
    eb2_ref[0] = eb[:, 2 * H:3 * H]
    eb2_ref[1] = eb[:, 3 * H:4 * H]


def _mid_body(xt_ref, agg_ref, deg_ref, wu_ref, bu_ref, g_ref, bln_ref,
              wmtT_ref, bm_ref, xt2_ref, a2_ref):
    agg = jnp.concatenate([agg_ref[0], agg_ref[1]], axis=1)
    deg = jnp.clip(deg_ref[0, :, 0:1] + deg_ref[1, :, 0:1], 1.0, None)
    out_t = jnp.dot(xt_ref[...] + agg / deg, wu_ref[...],
                    preferred_element_type=jnp.float32) + bu_ref[...]
    z = jax.nn.relu(_layernorm(out_t, g_ref[...], bln_ref[...]))
    nz = _norm(z)
    r = _artanh(jnp.minimum(jnp.tanh(nz), MAXNORM)) / nz
    xt2_ref[...] = r * z
    a2 = r * jnp.dot(z, wmtT_ref[...], preferred_element_type=jnp.float32) \
        + bm_ref[...]
    a2_ref[0] = a2[:, :H]
    a2_ref[1] = a2[:, H:]


def _final_body(xt_ref, agg_ref, deg_ref, wu_ref, bu_ref, g_ref,
                bln_ref, out_ref):
    agg = jnp.concatenate([agg_ref[0], agg_ref[1]], axis=1)
    deg = jnp.clip(deg_ref[0, :, 0:1] + deg_ref[1, :, 0:1], 1.0, None)
    out_t = jnp.dot(xt_ref[...] + agg / deg, wu_ref[...],
                    preferred_element_type=jnp.float32) + bu_ref[...]
    z = jax.nn.relu(_layernorm(out_t, g_ref[...], bln_ref[...]))
    nz = _norm(z)
    out_ref[...] = (jnp.minimum(jnp.tanh(nz), MAXNORM) / nz) * z


def _row_spec(r, cols):
    return pl.BlockSpec((r, cols), lambda i: (i, 0))

def _full_spec(shape):
    nd = len(shape)
    return pl.BlockSpec(shape, lambda i: (0,) * nd)

def _half_spec(r, w=H):
    return pl.BlockSpec((2, r, w), lambda i: (0, i, 0))


@functools.lru_cache(maxsize=None)
def _make_sc_layer(with_deg, n, e):
    edges_per_sub = e // SUBCORES
    ch = 80 if edges_per_sub % 80 == 0 else edges_per_sub
    chunks = edges_per_sub // ch
    edges_per_w = e // (2 * SUBCORES)
    ch2 = max(d for d in range(8, 129, 8) if edges_per_w % d == 0)
    chunks2 = edges_per_w // ch2
    zch = 32
    rows_per_sub = -(-(n // SUBCORES) // zch) * zch
    n_pad = rows_per_sub * SUBCORES
    zreps = rows_per_sub // zch

    mesh = plsc.VectorSubcoreMesh(core_axis_name="c", subcore_axis_name="s",
                                  num_cores=2, num_subcores=SUBCORES)

    agg_t = jax.ShapeDtypeStruct((2 * n_pad, H), jnp.float32)
    scratch = [
        pltpu.VMEM((ch,), jnp.int32),
        pltpu.VMEM((ch,), jnp.int32),
        pltpu.VMEM((ch,), jnp.int32),
        pltpu.VMEM((ch, H), jnp.float32),
        pltpu.VMEM((ch, H), jnp.float32),
        pltpu.VMEM((ch, H), jnp.float32),
        pltpu.VMEM((zch, H), jnp.float32),
        pltpu.VMEM_SHARED((n_pad, H), jnp.float32),
        pltpu.SemaphoreType.DMA,
        pltpu.SemaphoreType.DMA,
        pltpu.SemaphoreType.DMA,
        pltpu.SemaphoreType.DMA,
    ]
    if with_deg:
        scratch += [
            pltpu.VMEM((ch2,), jnp.int32),
            pltpu.VMEM((ch2, H), jnp.float32),
        ]

    @functools.partial(
        pl.kernel,
        out_type=(agg_t, agg_t) if with_deg else agg_t,
        mesh=mesh,
        scratch_types=scratch,
    )
    def sc_layer(acat, ebcat, src, dst, *refs):
        if with_deg:
            (out, outd, src_v, dst_v, gidx_v, a_v, eb_v, msg_v, z_v, agg_sp,
             sem_src, sem_dst, sem_eb, sem_g, dst2_v, ones_v) = refs
        else:
            (out, src_v, dst_v, gidx_v, a_v, eb_v, msg_v, z_v, agg_sp,
             sem_src, sem_dst, sem_eb, sem_g) = refs
        c = lax.axis_index("c")
        s = lax.axis_index("s")
        row0 = s * rows_per_sub
        cN = c * n
        cNp = c * n_pad
        cE = c * e

        def zrow(r, carry):
            for j in range(H // 16):
                z_v[r, pl.ds(j * 16, 16)] = jnp.zeros((16,), jnp.float32)
            return carry
        lax.fori_loop(0, zch, zrow, 0)

        def zcopy(t, carry):
            pltpu.sync_copy(z_v, agg_sp.at[pl.ds(row0 + t * zch, zch)])
            return carry
        lax.fori_loop(0, zreps, zcopy, 0)

        plsc.subcore_barrier()

        def chunk(k, carry):
            base = s * edges_per_sub + k * ch
            h_eb = pltpu.async_copy(ebcat.at[pl.ds(cE + base, ch)], eb_v,
                                    sem_eb)
            h_dst = pltpu.async_copy(dst.at[pl.ds(base, ch)], dst_v, sem_dst)
            h_src = pltpu.async_copy(src.at[pl.ds(base, ch)], src_v, sem_src)
            h_src.wait()
            for j in range(ch // 16):
                sl = pl.ds(j * 16, 16)
                gidx_v[sl] = src_v[sl] + cN
            h_g = pltpu.async_copy(acat.at[gidx_v], a_v, sem_g)
            h_g.wait()
            h_eb.wait()

            def mrow(r, rc):
                for j in range(H // 16):
                    sl = pl.ds(j * 16, 16)
                    msg_v[r, sl] = jnp.maximum(a_v[r, sl] + eb_v[r, sl], 0.0)
                return rc
            lax.fori_loop(0, ch, mrow, 0)

            h_dst.wait()
            pltpu.sync_copy(msg_v, agg_sp.at[dst_v], add=True)
            return carry

        lax.fori_loop(0, chunks, chunk, 0)

        plsc.subcore_barrier()
        pltpu.sync_copy(agg_sp.at[pl.ds(row0, rows_per_sub)],
                        out.at[pl.ds(cNp + row0, rows_per_sub)])

        if with_deg:
            lax.fori_loop(0, zreps, zcopy, 0)

            def onerow(r, carry):
                for j in range(H // 16):
                    ones_v[r, pl.ds(j * 16, 16)] = jnp.ones((16,), jnp.float32)
                return carry
            lax.fori_loop(0, ch2, onerow, 0)

            plsc.subcore_barrier()
            w0 = (c * SUBCORES + s) * edges_per_w

            def dchunk(k, carry):
                pltpu.sync_copy(dst.at[pl.ds(w0 + k * ch2, ch2)], dst2_v)
                pltpu.sync_copy(ones_v, agg_sp.at[dst2_v], add=True)
                return carry
            lax.fori_loop(0, chunks2, dchunk, 0)

            plsc.subcore_barrier()
            pltpu.sync_copy(agg_sp.at[pl.ds(row0, rows_per_sub)],
                            outd.at[pl.ds(cNp + row0, rows_per_sub)])

    return sc_layer


def kernel(x, adj, e, Wn, bn, We, be, Wm1, bm1, Wu1, bu1, g1, bln1,
           Wm2, bm2, Wu2, bu2, g2, bln2):
    f32 = jnp.float32
    n = x.shape[0]
    ne = e.shape[0]
    rn = 1000 if n % 1000 == 0 else n
    re_ = 640 if ne % 640 == 0 else ne
    n_pad = (-(-(n // SUBCORES) // 32) * 32) * SUBCORES

    src = adj[0]
    dst = adj[1]

    wm1t = Wm1[:D]
    wm2t = Wm2[:D]
    wmbot = jnp.concatenate([Wm1[D:], Wm2[D:]], axis=1)
    wnT = Wn.T
    weT = We.T
    w1n = jnp.concatenate([wnT, wnT @ wm1t], axis=1)
    w1e = jnp.concatenate([weT, weT @ wmbot], axis=1).astype(jnp.bfloat16)
    hb_n = _expmap0(bn.reshape(1, D))
    hb_e = _expmap0(be.reshape(1, D))
    hbw_n = hb_n @ wm1t
    hbw_e = hb_e @ wmbot
    bm1_r = bm1.reshape(1, D)
    bm2_r = bm2.reshape(1, D)
    bu1_r = bu1.reshape(1, D)
    bu2_r = bu2.reshape(1, D)
    g1_r = g1.reshape(1, D)
    g2_r = g2.reshape(1, D)
    bln1_r = bln1.reshape(1, D)
    bln2_r = bln2.reshape(1, D)

    xt1, a1 = pl.pallas_call(
        _node_pre_body,
        grid=(n // rn,),
        in_specs=[_row_spec(rn, D), _full_spec((D, 2 * D)), _full_spec((1, D)),
                  _full_spec((1, D)), _full_spec((1, D))],
        out_specs=[_row_spec(rn, D), _half_spec(rn)],
        out_shape=[jax.ShapeDtypeStruct((n, D), f32),
                   jax.ShapeDtypeStruct((2, n, H), f32)],
    )(x, w1n, hb_n, hbw_n, bm1_r)

    eb1, eb2 = pl.pallas_call(
        _edge_pre_body,
        grid=(ne // re_,),
        in_specs=[_row_spec(re_, D), _full_spec((D, 3 * D)), _full_spec((1, D)),
                  _full_spec((1, 2 * D))],
        out_specs=[_half_spec(re_), _half_spec(re_)],
        out_shape=[jax.ShapeDtypeStruct((2, ne, H), f32),
                   jax.ShapeDtypeStruct((2, ne, H), f32)],
    )(e, w1e, hb_e, hbw_e)

    agg1, deg1 = _make_sc_layer(True, n, ne)(
        a1.reshape(2 * n, H), eb1.reshape(2 * ne, H), src, dst)
    agg1 = agg1.reshape(2, n_pad, H)
    deg1 = deg1.reshape(2, n_pad, H)

    xt2, a2 = pl.pallas_call(
        _mid_body,
        grid=(n // rn,),
        in_specs=[_row_spec(rn, D), _half_spec(rn, H), _half_spec(rn, H),
                  _full_spec((D, D)), _full_spec((1, D)), _full_spec((1, D)),
                  _full_spec((1, D)), _full_spec((D, D)), _full_spec((1, D))],
        out_specs=[_row_spec(rn, D), _half_spec(rn)],
        out_shape=[jax.ShapeDtypeStruct((n, D), f32),
                   jax.ShapeDtypeStruct((2, n, H), f32)],
    )(xt1, agg1, deg1, Wu1, bu1_r, g1_r, bln1_r, wm2t, bm2_r)

    agg2 = _make_sc_layer(False, n, ne)(
        a2.reshape(2 * n, H), eb2.reshape(2 * ne, H), src, dst)
    agg2 = agg2.reshape(2, n_pad, H)

    out = pl.pallas_call(
        _final_body,
        grid=(n // rn,),
        in_specs=[_row_spec(rn, D), _half_spec(rn, H), _half_spec(rn, H),
                  _full_spec((D, D)), _full_spec((1, D)), _full_spec((1, D)),
                  _full_spec((1, D))],
        out_specs=_row_spec(rn, D),
        out_shape=jax.ShapeDtypeStruct((n, D), f32),
    )(xt2, agg2, deg1, Wu2, bu2_r, g2_r, bln2_r)

    return out
